# Initial kernel scaffold; baseline (speedup 1.0000x reference)
import jax
import jax.numpy as jnp
from jax.experimental import pallas as pl


def kernel(x, pos_x, pos_y, y, W, b, gamma, beta):
    raise NotImplementedError("write your pallas kernel here")


# speedup vs baseline: 11.6032x; 11.6032x over previous
import jax
import jax.numpy as jnp
from jax.experimental import pallas as pl
from jax.experimental.pallas import tpu as pltpu

_BQ = 256
_EPS_W = 1e-16
_EPS_BN = 1e-5


def _rne_bf16_f32(v):
    r = jax.lax.bitcast_convert_type(v, jnp.uint32)
    r = (r + 0x7FFF + ((r >> 16) & 1)) & jnp.uint32(0xFFFF0000)
    return jax.lax.bitcast_convert_type(r, jnp.float32)


def _fused_knn_linear_kernel(pxt_ref, x_ref, w_ref, b_ref, py_ref, y_ref,
                             h_ref, s_ref, q_ref):
    B = py_ref.shape[0]
    nx = pxt_ref.shape[1]
    py = py_ref[...]
    pxt = pxt_ref[...]
    sy = (py[:, 0:1] * py[:, 0:1] + py[:, 1:2] * py[:, 1:2]
          + py[:, 2:3] * py[:, 2:3])
    sx = (pxt[0:1, :] * pxt[0:1, :] + pxt[1:2, :] * pxt[1:2, :]
          + pxt[2:3, :] * pxt[2:3, :])
    pyb = _rne_bf16_f32(py)
    pxb = _rne_bf16_f32(pxt)
    dot = (pyb[:, 0:1] * pxb[0:1, :] + pyb[:, 1:2] * pxb[1:2, :]
           + pyb[:, 2:3] * pxb[2:3, :])
    d2 = (sy + sx) - 2.0 * dot
    lane = jax.lax.broadcasted_iota(jnp.int32, (B, nx), 1)
    oh = jnp.zeros((B, nx), jnp.float32)
    den = jnp.zeros((B, 1), jnp.float32)
    for _ in range(3):
        m = jnp.min(d2, axis=1, keepdims=True)
        sel = d2 == m
        idx = jnp.min(jnp.where(sel, lane, nx), axis=1, keepdims=True)
        w = 1.0 / jnp.maximum(m, _EPS_W)
        hit = lane == idx
        oh = oh + jnp.where(hit, w, 0.0)
        den = den + w
        d2 = jnp.where(hit, jnp.inf, d2)
    ohn = oh / den
    interp = jnp.dot(ohn, x_ref[...], preferred_element_type=jnp.float32)
    h = (jnp.dot(interp, w_ref[0:256, :], preferred_element_type=jnp.float32)
         + jnp.dot(y_ref[...], w_ref[256:512, :],
                   preferred_element_type=jnp.float32)
         + b_ref[...])
    h_ref[...] = h
    s_ref[0, 0, :] = jnp.sum(h, axis=0)
    q_ref[0, 0, :] = jnp.sum(h * h, axis=0)


def _bn_relu_kernel(s_ref, q_ref, g_ref, beta_ref, h_ref, o_ref, *, n):
    inv_n = 1.0 / n
    mean = jnp.sum(s_ref[...], axis=0, keepdims=True) * inv_n
    msq = jnp.sum(q_ref[...], axis=0, keepdims=True) * inv_n
    var = msq - mean * mean
    rstd = jax.lax.rsqrt(var + _EPS_BN)
    h = h_ref[...]
    o = (h - mean) * (rstd * g_ref[...]) + beta_ref[...]
    o_ref[...] = jnp.maximum(o, 0.0)


def kernel(x, pos_x, pos_y, y, W, b, gamma, beta):
    n_y = pos_y.shape[0]
    n_x = pos_x.shape[0]
    c_out = W.shape[1]
    nblk = n_y // _BQ

    pxt = pos_x.T
    b2 = b.reshape(1, c_out)
    g2 = gamma.reshape(1, c_out)
    be2 = beta.reshape(1, c_out)

    h, s, q = pl.pallas_call(
        _fused_knn_linear_kernel,
        grid=(nblk,),
        in_specs=[
            pl.BlockSpec((3, n_x), lambda i: (0, 0)),
            pl.BlockSpec(x.shape, lambda i: (0, 0)),
            pl.BlockSpec(W.shape, lambda i: (0, 0)),
            pl.BlockSpec((1, c_out), lambda i: (0, 0)),
            pl.BlockSpec((_BQ, 3), lambda i: (i, 0)),
            pl.BlockSpec((_BQ, y.shape[1]), lambda i: (i, 0)),
        ],
        out_specs=[
            pl.BlockSpec((_BQ, c_out), lambda i: (i, 0)),
            pl.BlockSpec((1, 1, c_out), lambda i: (i, 0, 0)),
            pl.BlockSpec((1, 1, c_out), lambda i: (i, 0, 0)),
        ],
        out_shape=[
            jax.ShapeDtypeStruct((n_y, c_out), jnp.float32),
            jax.ShapeDtypeStruct((nblk, 1, c_out), jnp.float32),
            jax.ShapeDtypeStruct((nblk, 1, c_out), jnp.float32),
        ],
    )(pxt, x, W, b2, pos_y, y)

    s2 = s.reshape(nblk, c_out)
    q2 = q.reshape(nblk, c_out)
    import functools
    out = pl.pallas_call(
        functools.partial(_bn_relu_kernel, n=float(n_y)),
        grid=(nblk,),
        in_specs=[
            pl.BlockSpec(s2.shape, lambda i: (0, 0)),
            pl.BlockSpec(q2.shape, lambda i: (0, 0)),
            pl.BlockSpec((1, c_out), lambda i: (0, 0)),
            pl.BlockSpec((1, c_out), lambda i: (0, 0)),
            pl.BlockSpec((_BQ, c_out), lambda i: (i, 0)),
        ],
        out_specs=pl.BlockSpec((_BQ, c_out), lambda i: (i, 0)),
        out_shape=jax.ShapeDtypeStruct((n_y, c_out), jnp.float32),
    )(s2, q2, g2, be2, h)
    return out


# speedup vs baseline: 12.8477x; 1.1073x over previous
import jax
import jax.numpy as jnp
from jax.experimental import pallas as pl
from jax.experimental.pallas import tpu as pltpu

_BQ = 256
_EPS_W = 1e-16
_EPS_BN = 1e-5


def _fused_knn_linear_kernel(pxt_ref, x_ref, w_ref, b_ref, py_ref, y_ref,
                             h_ref, s_ref, q_ref):
    B = py_ref.shape[0]
    nx = pxt_ref.shape[1]
    py = py_ref[...]
    pxt = pxt_ref[...]
    sy = (py[:, 0:1] * py[:, 0:1] + py[:, 1:2] * py[:, 1:2]
          + py[:, 2:3] * py[:, 2:3])
    sx = (pxt[0:1, :] * pxt[0:1, :] + pxt[1:2, :] * pxt[1:2, :]
          + pxt[2:3, :] * pxt[2:3, :])
    dot = jnp.dot(py.astype(jnp.bfloat16), pxt.astype(jnp.bfloat16),
                  preferred_element_type=jnp.float32)
    d2 = (sy + sx) - 2.0 * dot
    lane = jax.lax.broadcasted_iota(jnp.int32, (B, nx), 1)
    oh = jnp.zeros((B, nx), jnp.float32)
    den = jnp.zeros((B, 1), jnp.float32)
    for _ in range(3):
        m = jnp.min(d2, axis=1, keepdims=True)
        idx = jnp.min(jnp.where(d2 == m, lane, nx), axis=1, keepdims=True)
        w = 1.0 / jnp.maximum(m, _EPS_W)
        hit = lane == idx
        oh = jnp.where(hit, w, oh)
        den = den + w
        d2 = jnp.where(hit, jnp.inf, d2)
    ohn = oh / den
    interp = jnp.dot(ohn, x_ref[...], preferred_element_type=jnp.float32)
    h = (jnp.dot(interp, w_ref[0:256, :], preferred_element_type=jnp.float32)
         + jnp.dot(y_ref[...], w_ref[256:512, :],
                   preferred_element_type=jnp.float32)
         + b_ref[...])
    h_ref[...] = h
    s_ref[0, 0, :] = jnp.sum(h, axis=0)
    q_ref[0, 0, :] = jnp.sum(h * h, axis=0)


def _bn_relu_kernel(s_ref, q_ref, g_ref, beta_ref, h_ref, o_ref, *, n):
    inv_n = 1.0 / n
    mean = jnp.sum(s_ref[...], axis=0, keepdims=True) * inv_n
    msq = jnp.sum(q_ref[...], axis=0, keepdims=True) * inv_n
    var = msq - mean * mean
    rstd = jax.lax.rsqrt(var + _EPS_BN)
    h = h_ref[...]
    o = (h - mean) * (rstd * g_ref[...]) + beta_ref[...]
    o_ref[...] = jnp.maximum(o, 0.0)


def kernel(x, pos_x, pos_y, y, W, b, gamma, beta):
    n_y = pos_y.shape[0]
    n_x = pos_x.shape[0]
    c_out = W.shape[1]
    nblk = n_y // _BQ

    pxt = jnp.pad(pos_x.T, ((0, 5), (0, 0)))
    py8 = jnp.pad(pos_y, ((0, 0), (0, 5)))
    b2 = b.reshape(1, c_out)
    g2 = gamma.reshape(1, c_out)
    be2 = beta.reshape(1, c_out)

    h, s, q = pl.pallas_call(
        _fused_knn_linear_kernel,
        grid=(nblk,),
        in_specs=[
            pl.BlockSpec((8, n_x), lambda i: (0, 0)),
            pl.BlockSpec(x.shape, lambda i: (0, 0)),
            pl.BlockSpec(W.shape, lambda i: (0, 0)),
            pl.BlockSpec((1, c_out), lambda i: (0, 0)),
            pl.BlockSpec((_BQ, 8), lambda i: (i, 0)),
            pl.BlockSpec((_BQ, y.shape[1]), lambda i: (i, 0)),
        ],
        out_specs=[
            pl.BlockSpec((_BQ, c_out), lambda i: (i, 0)),
            pl.BlockSpec((1, 1, c_out), lambda i: (i, 0, 0)),
            pl.BlockSpec((1, 1, c_out), lambda i: (i, 0, 0)),
        ],
        out_shape=[
            jax.ShapeDtypeStruct((n_y, c_out), jnp.float32),
            jax.ShapeDtypeStruct((nblk, 1, c_out), jnp.float32),
            jax.ShapeDtypeStruct((nblk, 1, c_out), jnp.float32),
        ],
    )(pxt, x, W, b2, py8, y)

    s2 = s.reshape(nblk, c_out)
    q2 = q.reshape(nblk, c_out)
    import functools
    out = pl.pallas_call(
        functools.partial(_bn_relu_kernel, n=float(n_y)),
        grid=(nblk,),
        in_specs=[
            pl.BlockSpec(s2.shape, lambda i: (0, 0)),
            pl.BlockSpec(q2.shape, lambda i: (0, 0)),
            pl.BlockSpec((1, c_out), lambda i: (0, 0)),
            pl.BlockSpec((1, c_out), lambda i: (0, 0)),
            pl.BlockSpec((_BQ, c_out), lambda i: (i, 0)),
        ],
        out_specs=pl.BlockSpec((_BQ, c_out), lambda i: (i, 0)),
        out_shape=jax.ShapeDtypeStruct((n_y, c_out), jnp.float32),
    )(s2, q2, g2, be2, h)
    return out
